# baseline (speedup 1.0000x reference)
import functools

import jax
import jax.numpy as jnp
from jax import lax
from jax.experimental import pallas as pl
from jax.experimental.pallas import tpu as pltpu
from jax.experimental.pallas import tpu_sc as plsc

NC = 2
NS = 16
LANES = 16
NW = NC * NS

BATCH_N = 16384
HIST_N = 200
EDIM = 4
ODIM = 2

ROWW = 16
DLEN = 128
RPW = BATCH_N // NW
CB = 16
NCHUNK = RPW // CB
IDX_N = CB * HIST_N
NDESC = IDX_N // DLEN


def _sc_kernel_body(idx_hbm, table_hbm, w_hbm, b_hbm, out_hbm,
                    idx_v0, idx_v1, rows_v0, rows_v1, h_v, out_v, w_v, b_v,
                    sem0, sem1):
    wid = lax.axis_index("s") * NC + lax.axis_index("c")
    base_row = wid * RPW

    pltpu.sync_copy(w_hbm, w_v)
    pltpu.sync_copy(b_hbm, b_v)

    iota = lax.iota(jnp.int32, LANES)
    quad = iota >> 2
    col = iota & 3
    rbase = quad * HIST_N

    half = iota >> 1
    jout = iota & 1
    neg_inf = jnp.full((LANES,), -jnp.inf, dtype=jnp.float32)

    wv = [plsc.load_gather(w_v, [jout * EDIM + d]) for d in range(EDIM)]
    bv = plsc.load_gather(b_v, [jout])

    def issue(c, idx_v, rows_v, sem):
        row0 = ((base_row + c * CB) * HIST_N) // DLEN
        pltpu.sync_copy(idx_hbm.at[pl.ds(row0, NDESC)], idx_v)

        def mask_body(m, _):
            j = m // (DLEN // 16)
            k = m % (DLEN // 16)
            vec = idx_v[j, pl.ds(k * 16, 16)]
            idx_v[j, pl.ds(k * 16, 16)] = vec & 1023
            return 0

        lax.fori_loop(0, NDESC * (DLEN // 16), mask_body, 0)
        for j in range(NDESC):
            pltpu.async_copy(
                table_hbm.at[idx_v.at[j]],
                rows_v.at[pl.ds(j * DLEN, DLEN)],
                sem,
            )

    def drain(rows_v, sem):
        pltpu.make_async_copy(
            table_hbm.at[pl.ds(0, IDX_N)], rows_v, sem).wait()

    def compute(c, rows_v):
        def group_body(g, _):
            rb = rbase + g * (4 * HIST_N)

            def t_body(t, acc):
                v = plsc.load_gather(rows_v, [rb + t, col])
                return jnp.maximum(acc, v)

            acc = lax.fori_loop(0, HIST_N, t_body, neg_inf, unroll=8)
            h_v[pl.ds((c * CB + g * 4) * EDIM, LANES)] = acc
            return 0

        lax.fori_loop(0, CB // 4, group_body, 0)

    issue(0, idx_v0, rows_v0, sem0)

    def pair_body(i, _):
        c = i * 2
        issue(c + 1, idx_v1, rows_v1, sem1)
        drain(rows_v0, sem0)
        compute(c, rows_v0)

        @pl.when(c + 2 < NCHUNK)
        def _():
            issue(c + 2, idx_v0, rows_v0, sem0)

        drain(rows_v1, sem1)
        compute(c + 1, rows_v1)
        return 0

    lax.fori_loop(0, NCHUNK // 2, pair_body, 0)

    def out_body(o, _):
        hbase = (o * 8 + half) * EDIM
        acc = bv
        for d in range(EDIM):
            acc = acc + wv[d] * plsc.load_gather(h_v, [hbase + d])
        out_v[pl.ds(o * LANES, LANES)] = acc
        return 0

    lax.fori_loop(0, RPW * ODIM // LANES, out_body, 0)
    pltpu.sync_copy(out_v, out_hbm.at[pl.ds(base_row * ODIM, RPW * ODIM)])


@functools.partial(jax.jit, static_argnames=())
def kernel(inputs, embed_table, W, b):
    idx_flat = inputs.reshape(-1, DLEN).astype(jnp.int32)
    w_flat = W.reshape(-1).astype(jnp.float32)
    b_pad = jnp.zeros((8,), jnp.float32).at[:ODIM].set(b)
    table_pad = jnp.pad(embed_table, ((0, 0), (0, ROWW - EDIM)))

    mesh = plsc.VectorSubcoreMesh(core_axis_name="c", subcore_axis_name="s")
    run = pl.kernel(
        _sc_kernel_body,
        out_type=jax.ShapeDtypeStruct((BATCH_N * ODIM,), jnp.float32),
        mesh=mesh,
        scratch_types=[
            pltpu.VMEM((NDESC, DLEN), jnp.int32),
            pltpu.VMEM((NDESC, DLEN), jnp.int32),
            pltpu.VMEM((IDX_N, ROWW), jnp.float32),
            pltpu.VMEM((IDX_N, ROWW), jnp.float32),
            pltpu.VMEM((RPW * EDIM,), jnp.float32),
            pltpu.VMEM((RPW * ODIM,), jnp.float32),
            pltpu.VMEM((ODIM * EDIM,), jnp.float32),
            pltpu.VMEM((8,), jnp.float32),
            pltpu.SemaphoreType.DMA,
            pltpu.SemaphoreType.DMA,
        ],
        compiler_params=pltpu.CompilerParams(
            needs_layout_passes=False, use_tc_tiling_on_sc=False),
    )
    out = run(idx_flat, table_pad, w_flat, b_pad)
    return out.reshape(BATCH_N, ODIM)


# speedup vs baseline: 1.1297x; 1.1297x over previous
import functools

import jax
import jax.numpy as jnp
from jax import lax
from jax.experimental import pallas as pl
from jax.experimental.pallas import tpu as pltpu
from jax.experimental.pallas import tpu_sc as plsc

NC = 2
NS = 16
LANES = 16
NW = NC * NS

BATCH_N = 16384
HIST_N = 200
EDIM = 4
ODIM = 2

ROWW = 16
DLEN = 128
RPW = BATCH_N // NW
CB = 16
NCHUNK = RPW // CB
IDX_N = CB * HIST_N
NDESC = IDX_N // DLEN


def _sc_kernel_body(idx_hbm, table_hbm, w_hbm, b_hbm, out_hbm,
                    idx_v0, idx_v1, rows_v0, rows_v1, h_v, out_v, w_v, b_v,
                    sem0, sem1):
    wid = lax.axis_index("s") * NC + lax.axis_index("c")
    base_row = wid * RPW

    pltpu.sync_copy(w_hbm, w_v)
    pltpu.sync_copy(b_hbm, b_v)

    iota = lax.iota(jnp.int32, LANES)
    quad = iota >> 2
    col = iota & 3
    rbase = quad * HIST_N

    half = iota >> 1
    jout = iota & 1
    neg_inf = jnp.full((LANES,), -jnp.inf, dtype=jnp.float32)

    wv = [plsc.load_gather(w_v, [jout * EDIM + d]) for d in range(EDIM)]
    bv = plsc.load_gather(b_v, [jout])

    def issue(c, idx_v, rows_v, sem):
        row0 = ((base_row + c * CB) * HIST_N) // DLEN
        pltpu.sync_copy(idx_hbm.at[pl.ds(row0, NDESC)], idx_v)
        for j in range(NDESC):
            pltpu.async_copy(
                table_hbm.at[idx_v.at[j]],
                rows_v.at[pl.ds(j * DLEN, DLEN)],
                sem,
            )

    def drain(rows_v, sem):
        pltpu.make_async_copy(
            table_hbm.at[pl.ds(0, IDX_N)], rows_v, sem).wait()

    def compute(c, rows_v):
        def group_body(g, _):
            rb = rbase + g * (4 * HIST_N)

            def t_body(t, acc):
                v = plsc.load_gather(rows_v, [rb + t, col])
                return jnp.maximum(acc, v)

            acc = lax.fori_loop(0, HIST_N, t_body, neg_inf, unroll=8)
            h_v[pl.ds((c * CB + g * 4) * EDIM, LANES)] = acc
            return 0

        lax.fori_loop(0, CB // 4, group_body, 0)

    issue(0, idx_v0, rows_v0, sem0)

    def pair_body(i, _):
        c = i * 2
        issue(c + 1, idx_v1, rows_v1, sem1)
        drain(rows_v0, sem0)
        compute(c, rows_v0)

        @pl.when(c + 2 < NCHUNK)
        def _():
            issue(c + 2, idx_v0, rows_v0, sem0)

        drain(rows_v1, sem1)
        compute(c + 1, rows_v1)
        return 0

    lax.fori_loop(0, NCHUNK // 2, pair_body, 0)

    def out_body(o, _):
        hbase = (o * 8 + half) * EDIM
        acc = bv
        for d in range(EDIM):
            acc = acc + wv[d] * plsc.load_gather(h_v, [hbase + d])
        out_v[pl.ds(o * LANES, LANES)] = acc
        return 0

    lax.fori_loop(0, RPW * ODIM // LANES, out_body, 0)
    pltpu.sync_copy(out_v, out_hbm.at[pl.ds(base_row * ODIM, RPW * ODIM)])


@functools.partial(jax.jit, static_argnames=())
def kernel(inputs, embed_table, W, b):
    idx_flat = inputs.reshape(-1, DLEN).astype(jnp.int32)
    w_flat = W.reshape(-1).astype(jnp.float32)
    b_pad = jnp.zeros((8,), jnp.float32).at[:ODIM].set(b)
    table_pad = jnp.pad(embed_table, ((0, 0), (0, ROWW - EDIM)))

    mesh = plsc.VectorSubcoreMesh(core_axis_name="c", subcore_axis_name="s")
    run = pl.kernel(
        _sc_kernel_body,
        out_type=jax.ShapeDtypeStruct((BATCH_N * ODIM,), jnp.float32),
        mesh=mesh,
        scratch_types=[
            pltpu.VMEM((NDESC, DLEN), jnp.int32),
            pltpu.VMEM((NDESC, DLEN), jnp.int32),
            pltpu.VMEM((IDX_N, ROWW), jnp.float32),
            pltpu.VMEM((IDX_N, ROWW), jnp.float32),
            pltpu.VMEM((RPW * EDIM,), jnp.float32),
            pltpu.VMEM((RPW * ODIM,), jnp.float32),
            pltpu.VMEM((ODIM * EDIM,), jnp.float32),
            pltpu.VMEM((8,), jnp.float32),
            pltpu.SemaphoreType.DMA,
            pltpu.SemaphoreType.DMA,
        ],
        compiler_params=pltpu.CompilerParams(
            needs_layout_passes=False, use_tc_tiling_on_sc=False),
    )
    out = run(idx_flat, table_pad, w_flat, b_pad)
    return out.reshape(BATCH_N, ODIM)


# speedup vs baseline: 1.2877x; 1.1399x over previous
import functools

import jax
import jax.numpy as jnp
from jax import lax
from jax.experimental import pallas as pl
from jax.experimental.pallas import tpu as pltpu
from jax.experimental.pallas import tpu_sc as plsc

NC = 2
NS = 16
LANES = 16
NW = NC * NS

BATCH_N = 16384
HIST_N = 200
EDIM = 4
ODIM = 2

ROWW = 16
DLEN = 128
RPW = BATCH_N // NW
CB = 16
NCHUNK = RPW // CB
IDX_N = CB * HIST_N
NDESC = IDX_N // DLEN


def _sc_kernel_body(idx_hbm, table_hbm, w_hbm, b_hbm, out_hbm,
                    idx_v0, idx_v1, rows_v0, rows_v1, lo_v0, lo_v1,
                    h_v, out_v, w_v, b_v, sem0, sem1):
    wid = lax.axis_index("s") * NC + lax.axis_index("c")
    base_row = wid * RPW

    pltpu.sync_copy(w_hbm, w_v)
    pltpu.sync_copy(b_hbm, b_v)

    iota = lax.iota(jnp.int32, LANES)
    quad = iota >> 2
    col = iota & 3
    rbase = quad * HIST_N

    half = iota >> 1
    jout = iota & 1
    neg_inf = jnp.full((LANES,), -jnp.inf, dtype=jnp.float32)

    wv = [plsc.load_gather(w_v, [jout * EDIM + d]) for d in range(EDIM)]
    bv = plsc.load_gather(b_v, [jout])

    def issue(c, idx_v, rows_v, lo_v, sem):
        row0 = ((base_row + c * CB) * HIST_N) // DLEN
        pltpu.sync_copy(idx_hbm.at[pl.ds(row0, NDESC)], idx_v)

        def split_body(m, _):
            j = m // (DLEN // 16)
            k = m % (DLEN // 16)
            vec = idx_v[j, pl.ds(k * 16, 16)]
            idx_v[j, pl.ds(k * 16, 16)] = vec >> 2
            lo_v[pl.ds(j * DLEN + k * 16, 16)] = (vec & 3) << 2
            return 0

        lax.fori_loop(0, NDESC * (DLEN // 16), split_body, 0, unroll=8)
        for j in range(NDESC):
            pltpu.async_copy(
                table_hbm.at[idx_v.at[j]],
                rows_v.at[pl.ds(j * DLEN, DLEN)],
                sem,
            )

    def drain(rows_v, sem):
        pltpu.make_async_copy(
            table_hbm.at[pl.ds(0, IDX_N)], rows_v, sem).wait()

    def compute(c, rows_v, lo_v):
        def group_body(g, _):
            rb = rbase + g * (4 * HIST_N)

            def t_body(t, acc):
                lo = plsc.load_gather(lo_v, [rb + t])
                v = plsc.load_gather(rows_v, [rb + t, lo + col])
                return jnp.maximum(acc, v)

            acc = lax.fori_loop(0, HIST_N, t_body, neg_inf, unroll=8)
            h_v[pl.ds((c * CB + g * 4) * EDIM, LANES)] = acc
            return 0

        lax.fori_loop(0, CB // 4, group_body, 0)

    issue(0, idx_v0, rows_v0, lo_v0, sem0)

    def pair_body(i, _):
        c = i * 2
        issue(c + 1, idx_v1, rows_v1, lo_v1, sem1)
        drain(rows_v0, sem0)
        compute(c, rows_v0, lo_v0)

        @pl.when(c + 2 < NCHUNK)
        def _():
            issue(c + 2, idx_v0, rows_v0, lo_v0, sem0)

        drain(rows_v1, sem1)
        compute(c + 1, rows_v1, lo_v1)
        return 0

    lax.fori_loop(0, NCHUNK // 2, pair_body, 0)

    def out_body(o, _):
        hbase = (o * 8 + half) * EDIM
        acc = bv
        for d in range(EDIM):
            acc = acc + wv[d] * plsc.load_gather(h_v, [hbase + d])
        out_v[pl.ds(o * LANES, LANES)] = acc
        return 0

    lax.fori_loop(0, RPW * ODIM // LANES, out_body, 0)
    pltpu.sync_copy(out_v, out_hbm.at[pl.ds(base_row * ODIM, RPW * ODIM)])


@functools.partial(jax.jit, static_argnames=())
def kernel(inputs, embed_table, W, b):
    idx_flat = inputs.reshape(-1, DLEN).astype(jnp.int32)
    w_flat = W.reshape(-1).astype(jnp.float32)
    b_pad = jnp.zeros((8,), jnp.float32).at[:ODIM].set(b)
    table_g = embed_table.reshape(-1, ROWW)

    mesh = plsc.VectorSubcoreMesh(core_axis_name="c", subcore_axis_name="s")
    run = pl.kernel(
        _sc_kernel_body,
        out_type=jax.ShapeDtypeStruct((BATCH_N * ODIM,), jnp.float32),
        mesh=mesh,
        scratch_types=[
            pltpu.VMEM((NDESC, DLEN), jnp.int32),
            pltpu.VMEM((NDESC, DLEN), jnp.int32),
            pltpu.VMEM((IDX_N, ROWW), jnp.float32),
            pltpu.VMEM((IDX_N, ROWW), jnp.float32),
            pltpu.VMEM((IDX_N,), jnp.int32),
            pltpu.VMEM((IDX_N,), jnp.int32),
            pltpu.VMEM((RPW * EDIM,), jnp.float32),
            pltpu.VMEM((RPW * ODIM,), jnp.float32),
            pltpu.VMEM((ODIM * EDIM,), jnp.float32),
            pltpu.VMEM((8,), jnp.float32),
            pltpu.SemaphoreType.DMA,
            pltpu.SemaphoreType.DMA,
        ],
        compiler_params=pltpu.CompilerParams(
            needs_layout_passes=False, use_tc_tiling_on_sc=False),
    )
    out = run(idx_flat, table_g, w_flat, b_pad)
    return out.reshape(BATCH_N, ODIM)


# speedup vs baseline: 1.2880x; 1.0002x over previous
import functools

import jax
import jax.numpy as jnp
from jax import lax
from jax.experimental import pallas as pl
from jax.experimental.pallas import tpu as pltpu
from jax.experimental.pallas import tpu_sc as plsc

NC = 2
NS = 16
LANES = 16
NW = NC * NS

BATCH_N = 16384
HIST_N = 200
EDIM = 4
ODIM = 2

ROWW = 16
DLEN = 128
RPW = BATCH_N // NW
CB = 16
NCHUNK = RPW // CB
IDX_N = CB * HIST_N
NDESC = IDX_N // DLEN


def _sc_kernel_body(idx_hbm, table_hbm, w_hbm, b_hbm, out_hbm,
                    idx_v0, idx_v1, rows_v0, rows_v1, lo_v0, lo_v1,
                    h_v, out_v, w_v, b_v, sem0, sem1):
    wid = lax.axis_index("s") * NC + lax.axis_index("c")
    base_row = wid * RPW

    pltpu.sync_copy(w_hbm, w_v)
    pltpu.sync_copy(b_hbm, b_v)

    iota = lax.iota(jnp.int32, LANES)
    quad = iota >> 2
    col = iota & 3
    rbase = quad * HIST_N

    half = iota >> 1
    jout = iota & 1
    neg_inf = jnp.full((LANES,), -jnp.inf, dtype=jnp.float32)

    wv = [plsc.load_gather(w_v, [jout * EDIM + d]) for d in range(EDIM)]
    bv = plsc.load_gather(b_v, [jout])

    def issue(c, idx_v, rows_v, lo_v, sem):
        row0 = ((base_row + c * CB) * HIST_N) // DLEN
        pltpu.sync_copy(idx_hbm.at[pl.ds(row0, NDESC)], idx_v)

        def split_body(m, _):
            j = m // (DLEN // 16)
            k = m % (DLEN // 16)
            vec = idx_v[j, pl.ds(k * 16, 16)]
            idx_v[j, pl.ds(k * 16, 16)] = vec >> 2
            lo_v[pl.ds(j * DLEN + k * 16, 16)] = (vec & 3) << 2
            return 0

        lax.fori_loop(0, NDESC * (DLEN // 16), split_body, 0, unroll=8)
        for j in range(NDESC):
            pltpu.async_copy(
                table_hbm.at[idx_v.at[j]],
                rows_v.at[pl.ds(j * DLEN, DLEN)],
                sem,
            )

    def drain(rows_v, sem):
        pltpu.make_async_copy(
            table_hbm.at[pl.ds(0, IDX_N)], rows_v, sem).wait()

    def compute(c, rows_v, lo_v):
        def group_body(g, _):
            rb = rbase + g * (4 * HIST_N)

            def t_body(t, acc):
                lo = plsc.load_gather(lo_v, [rb + t])
                v = plsc.load_gather(rows_v, [rb + t, lo + col])
                return jnp.maximum(acc, v)

            acc = lax.fori_loop(0, HIST_N, t_body, neg_inf, unroll=8)
            h_v[pl.ds((c * CB + g * 4) * EDIM, LANES)] = acc
            return 0

        lax.fori_loop(0, CB // 4, group_body, 0)

    issue(0, idx_v0, rows_v0, lo_v0, sem0)

    def pair_body(i, _):
        c = i * 2
        issue(c + 1, idx_v1, rows_v1, lo_v1, sem1)
        drain(rows_v0, sem0)
        compute(c, rows_v0, lo_v0)

        @pl.when(c + 2 < NCHUNK)
        def _():
            issue(c + 2, idx_v0, rows_v0, lo_v0, sem0)

        drain(rows_v1, sem1)
        compute(c + 1, rows_v1, lo_v1)
        return 0

    lax.fori_loop(0, NCHUNK // 2, pair_body, 0)

    def out_body(o, _):
        hbase = (o * 8 + half) * EDIM
        acc = bv
        for d in range(EDIM):
            acc = acc + wv[d] * plsc.load_gather(h_v, [hbase + d])
        out_v[pl.ds(o * LANES, LANES)] = acc
        return 0

    lax.fori_loop(0, RPW * ODIM // LANES, out_body, 0)
    pltpu.sync_copy(out_v, out_hbm.at[pl.ds(base_row * ODIM, RPW * ODIM)])


@jax.jit
def kernel(inputs, embed_table, W, b):
    idx_flat = inputs.reshape(-1, DLEN).astype(jnp.int32)
    w_flat = W.reshape(-1).astype(jnp.float32)
    b_pad = jnp.zeros((8,), jnp.float32).at[:ODIM].set(b)
    t128 = lax.optimization_barrier(embed_table.reshape(-1, 128))
    table_g = t128.reshape(-1, ROWW)

    mesh = plsc.VectorSubcoreMesh(core_axis_name="c", subcore_axis_name="s")
    run = pl.kernel(
        _sc_kernel_body,
        out_type=jax.ShapeDtypeStruct((BATCH_N * ODIM,), jnp.float32),
        mesh=mesh,
        scratch_types=[
            pltpu.VMEM((NDESC, DLEN), jnp.int32),
            pltpu.VMEM((NDESC, DLEN), jnp.int32),
            pltpu.VMEM((IDX_N, ROWW), jnp.float32),
            pltpu.VMEM((IDX_N, ROWW), jnp.float32),
            pltpu.VMEM((IDX_N,), jnp.int32),
            pltpu.VMEM((IDX_N,), jnp.int32),
            pltpu.VMEM((RPW * EDIM,), jnp.float32),
            pltpu.VMEM((RPW * ODIM,), jnp.float32),
            pltpu.VMEM((ODIM * EDIM,), jnp.float32),
            pltpu.VMEM((8,), jnp.float32),
            pltpu.SemaphoreType.DMA,
            pltpu.SemaphoreType.DMA,
        ],
        compiler_params=pltpu.CompilerParams(
            needs_layout_passes=False, use_tc_tiling_on_sc=False),
    )
    out = run(idx_flat, table_g, w_flat, b_pad)
    return out.reshape(BATCH_N, ODIM)


# speedup vs baseline: 1.2901x; 1.0016x over previous
import functools

import jax
import jax.numpy as jnp
from jax import lax
from jax.experimental import pallas as pl
from jax.experimental.pallas import tpu as pltpu
from jax.experimental.pallas import tpu_sc as plsc

NC = 2
NS = 16
LANES = 16
NW = NC * NS

BATCH_N = 16384
HIST_N = 200
EDIM = 4
ODIM = 2

ROWW = 16
DLEN = 128
RPW = BATCH_N // NW
CB = 16
NCHUNK = RPW // CB
IDX_N = CB * HIST_N
NDESC = IDX_N // DLEN


def _sc_kernel_body(idx_hbm, table_hbm, w_hbm, b_hbm, out_hbm,
                    idx_v0, idx_v1, rows_v0, rows_v1, lo_v0, lo_v1,
                    h_v, out_v, w_v, b_v, sem0, sem1):
    wid = lax.axis_index("s") * NC + lax.axis_index("c")
    base_row = wid * RPW

    pltpu.sync_copy(w_hbm, w_v)
    pltpu.sync_copy(b_hbm, b_v)

    iota = lax.iota(jnp.int32, LANES)
    quad = iota >> 2
    col = iota & 3
    rbase = quad * HIST_N

    half = iota >> 1
    jout = iota & 1
    neg_inf = jnp.full((LANES,), -jnp.inf, dtype=jnp.float32)

    wv = [plsc.load_gather(w_v, [jout * EDIM + d]) for d in range(EDIM)]
    bv = plsc.load_gather(b_v, [jout])

    def issue(c, idx_v, rows_v, lo_v, sem):
        row0 = ((base_row + c * CB) * HIST_N) // DLEN
        pltpu.sync_copy(idx_hbm.at[pl.ds(row0, NDESC)], idx_v)

        def split_body(m, _):
            j = m // (DLEN // 16)
            k = m % (DLEN // 16)
            vec = idx_v[j, pl.ds(k * 16, 16)]
            idx_v[j, pl.ds(k * 16, 16)] = vec >> 2
            lo_v[pl.ds(j * DLEN + k * 16, 16)] = (vec & 3) << 2
            return 0

        lax.fori_loop(0, NDESC * (DLEN // 16), split_body, 0, unroll=8)
        for j in range(NDESC):
            pltpu.async_copy(
                table_hbm.at[idx_v.at[j]],
                rows_v.at[pl.ds(j * DLEN, DLEN)],
                sem,
            )

    def drain(rows_v, sem):
        pltpu.make_async_copy(
            table_hbm.at[pl.ds(0, IDX_N)], rows_v, sem).wait()

    def compute(c, rows_v, lo_v):
        def group_body(g, _):
            rb = rbase + g * (4 * HIST_N)

            def t_body(t, acc):
                lo = plsc.load_gather(lo_v, [rb + t])
                v = plsc.load_gather(rows_v, [rb + t, lo + col])
                return jnp.maximum(acc, v)

            acc = lax.fori_loop(0, HIST_N, t_body, neg_inf, unroll=8)
            h_v[pl.ds((c * CB + g * 4) * EDIM, LANES)] = acc
            return 0

        lax.fori_loop(0, CB // 4, group_body, 0)

    issue(0, idx_v0, rows_v0, lo_v0, sem0)

    def pair_body(i, _):
        c = i * 2
        issue(c + 1, idx_v1, rows_v1, lo_v1, sem1)
        drain(rows_v0, sem0)
        compute(c, rows_v0, lo_v0)

        @pl.when(c + 2 < NCHUNK)
        def _():
            issue(c + 2, idx_v0, rows_v0, lo_v0, sem0)

        drain(rows_v1, sem1)
        compute(c + 1, rows_v1, lo_v1)
        return 0

    lax.fori_loop(0, NCHUNK // 2, pair_body, 0)

    def out_body(o, _):
        hbase = (o * 8 + half) * EDIM
        acc = bv
        for d in range(EDIM):
            acc = acc + wv[d] * plsc.load_gather(h_v, [hbase + d])
        out_v[pl.ds(o * LANES, LANES)] = acc
        return 0

    lax.fori_loop(0, RPW * ODIM // LANES, out_body, 0)
    pltpu.sync_copy(out_v, out_hbm.at[pl.ds(base_row * ODIM, RPW * ODIM)])


@jax.jit
def kernel(inputs, embed_table, W, b):
    idx_flat = lax.optimization_barrier(
        inputs.astype(jnp.int32).reshape(-1, DLEN))
    w_flat = W.reshape(-1).astype(jnp.float32)
    b_pad = jnp.zeros((8,), jnp.float32).at[:ODIM].set(b)
    table_g = embed_table.reshape(-1, ROWW)

    mesh = plsc.VectorSubcoreMesh(core_axis_name="c", subcore_axis_name="s")
    run = pl.kernel(
        _sc_kernel_body,
        out_type=jax.ShapeDtypeStruct((BATCH_N * ODIM,), jnp.float32),
        mesh=mesh,
        scratch_types=[
            pltpu.VMEM((NDESC, DLEN), jnp.int32),
            pltpu.VMEM((NDESC, DLEN), jnp.int32),
            pltpu.VMEM((IDX_N, ROWW), jnp.float32),
            pltpu.VMEM((IDX_N, ROWW), jnp.float32),
            pltpu.VMEM((IDX_N,), jnp.int32),
            pltpu.VMEM((IDX_N,), jnp.int32),
            pltpu.VMEM((RPW * EDIM,), jnp.float32),
            pltpu.VMEM((RPW * ODIM,), jnp.float32),
            pltpu.VMEM((ODIM * EDIM,), jnp.float32),
            pltpu.VMEM((8,), jnp.float32),
            pltpu.SemaphoreType.DMA,
            pltpu.SemaphoreType.DMA,
        ],
        compiler_params=pltpu.CompilerParams(
            needs_layout_passes=False, use_tc_tiling_on_sc=False),
    )
    out = run(idx_flat, table_g, w_flat, b_pad)
    return out.reshape(BATCH_N, ODIM)
